# Initial kernel scaffold; baseline (speedup 1.0000x reference)
import jax
import jax.numpy as jnp
from jax.experimental import pallas as pl


def kernel(node_feat, edge_feat, edge_index, enc_node_W, enc_node_b, enc_edge_W, enc_edge_b, ln_g, ln_b, conv_W, conv_b, conv_beta, ro_logit_W, ro_logit_b, ro_proj_W, ro_proj_b, gru_Wih, gru_Whh, gru_bih, gru_bhh, out_W1, out_b1, out_W2, out_b2):
    raise NotImplementedError("write your pallas kernel here")


# speedup vs baseline: 1.0000x; 1.0000x over previous
import jax, jax.numpy as jnp
import numpy as np
from jax.experimental import pallas as pl

N = 10000
E = 160000
H = 128
L = 7
T = 2
EPS = 1e-7


def _layer_norm(x, g, b):
    mu = jnp.mean(x, axis=-1, keepdims=True)
    var = jnp.var(x, axis=-1, keepdims=True)
    return (x - mu) / jnp.sqrt(var + 1e-5) * g + b


def _edge_softmax(logits, dst):
    mx = jax.ops.segment_max(logits, dst, num_segments=N)
    mx = jnp.where(jnp.isfinite(mx), mx, 0.0)
    ex = jnp.exp(logits - mx[dst])
    s = jax.ops.segment_sum(ex, dst, num_segments=N)
    return ex / s[dst]


def _identity_kernel(x_ref, o_ref):
    o_ref[...] = x_ref[...]


def kernel(node_feat, edge_feat, edge_index, enc_node_W, enc_node_b, enc_edge_W, enc_edge_b, ln_g, ln_b, conv_W, conv_b, conv_beta, ro_logit_W, ro_logit_b, ro_proj_W, ro_proj_b, gru_Wih, gru_Whh, gru_bih, gru_bhh, out_W1, out_b1, out_W2, out_b2):
    src = edge_index[0]
    dst = edge_index[1]
    hv = node_feat @ enc_node_W + enc_node_b
    he = edge_feat @ enc_edge_W + enc_edge_b
    for l in range(L):
        h1 = jax.nn.relu(_layer_norm(hv, ln_g[l], ln_b[l]))
        m = jax.nn.relu(h1[src] + he) + EPS
        a = _edge_softmax(m * conv_beta[l], dst)
        agg = jax.ops.segment_sum(m * a, dst, num_segments=N)
        hv = (h1 + agg) @ conv_W[l] + conv_b[l] + hv
    g_feats = jnp.sum(hv, axis=0)
    for t in range(T):
        ctx = jnp.broadcast_to(jax.nn.relu(g_feats), (N, H))
        z = jnp.concatenate([ctx, hv], axis=1) @ ro_logit_W[t] + ro_logit_b[t]
        z = jax.nn.leaky_relu(z, negative_slope=0.01)
        a = jax.nn.softmax(z)
        proj = hv @ ro_proj_W[t] + ro_proj_b[t]
        g_repr = jax.nn.elu(jnp.sum(a[:, None] * proj, axis=0))
        gi = gru_Wih[t] @ g_repr + gru_bih[t]
        gh = gru_Whh[t] @ g_feats + gru_bhh[t]
        r = jax.nn.sigmoid(gi[:H] + gh[:H])
        u = jax.nn.sigmoid(gi[H:2 * H] + gh[H:2 * H])
        c = jnp.tanh(gi[2 * H:] + r * gh[2 * H:])
        g_feats = jax.nn.relu((1.0 - u) * c + u * g_feats)
    out = jax.nn.relu(g_feats @ out_W1 + out_b1) @ out_W2 + out_b2
    out = pl.pallas_call(
        _identity_kernel,
        out_shape=jax.ShapeDtypeStruct(out.shape, out.dtype),
    )(out)
    return out


# speedup vs baseline: 1.9733x; 1.9732x over previous
import functools

import jax
import jax.numpy as jnp
from jax import lax
from jax.experimental import pallas as pl
from jax.experimental.pallas import tpu as pltpu
from jax.experimental.pallas import tpu_sc as plsc

N = 10000
E = 160000
DE = 16
H = 128
L = 7
T = 2
EPS = 1e-7

NSUB = 16
CH = 128
EP = 163840
NCHUNK = EP // CH
CPT = NCHUNK // NSUB
NP = 10112
RB = 400
EB = 2048

f32 = jnp.float32


@functools.cache
def _get_sc_kernel():
    mesh = plsc.VectorSubcoreMesh(core_axis_name="c", subcore_axis_name="s")

    @functools.partial(
        pl.kernel,
        out_type=jax.ShapeDtypeStruct((2, NP, 128), f32),
        mesh=mesh,
        scratch_types=[
            pltpu.VMEM((CH,), jnp.int32),
            pltpu.VMEM((CH,), jnp.int32),
            pltpu.VMEM((CH, 128), f32),
            pltpu.VMEM((CH, 128), f32),
            pltpu.VMEM((CH, 128), f32),
            pltpu.VMEM((16,), f32),
            pltpu.VMEM_SHARED((NP, 128), f32),
            pltpu.SemaphoreType.DMA,
        ],
    )
    def _sc_layer_agg(h1, hef, srci, dsti, betav,
                      acc_out,
                      idx_s, idx_d, gbuf, hebuf, vals, bbuf, acc, sem):
        cid = lax.axis_index("c")
        sid = lax.axis_index("s")

        zv = jnp.zeros((16,), f32)

        @pl.loop(0, 32)
        def _(i):
            for k in range(8):
                vals[i, pl.ds(k * 16, 16)] = zv

        @pl.loop(sid, NP // 32, step=NSUB)
        def _(c):
            pltpu.sync_copy(vals.at[pl.ds(0, 32)], acc.at[pl.ds(c * 32, 32)])

        pltpu.sync_copy(betav, bbuf)
        plsc.subcore_barrier()

        @pl.loop(sid * CPT, (sid + 1) * CPT)
        def _(r):
            base = r * CH
            pltpu.sync_copy(srci.at[pl.ds(base, CH)], idx_s)
            pltpu.sync_copy(dsti.at[pl.ds(base, CH)], idx_d)
            pltpu.async_copy(h1.at[idx_s], gbuf, sem).wait()
            pltpu.sync_copy(hef.at[pl.ds(base, CH)], hebuf)

            bv = bbuf[pl.ds(0, 16)]
            off = cid * 64

            @pl.loop(0, CH)
            def _(i):
                for k in range(4):
                    g = gbuf[i, pl.ds(off + k * 16, 16)]
                    hh = hebuf[i, pl.ds(off + k * 16, 16)]
                    m = jnp.maximum(g + hh, 0.0) + EPS
                    w = jnp.exp(m * bv)
                    vals[i, pl.ds(k * 16, 16)] = m * w
                    vals[i, pl.ds(64 + k * 16, 16)] = w

            pltpu.sync_copy(vals, acc.at[idx_d], add=True)

        plsc.subcore_barrier()

        @pl.loop(sid, NP // 32, step=NSUB)
        def _(c):
            pltpu.sync_copy(acc.at[pl.ds(c * 32, 32)], vals.at[pl.ds(0, 32)])
            pltpu.sync_copy(vals.at[pl.ds(0, 32)],
                            acc_out.at[cid, pl.ds(c * 32, 32), :])

    return _sc_layer_agg


def _ln_relu(x, g, b):
    mu = jnp.mean(x, axis=-1, keepdims=True)
    var = jnp.mean((x - mu) * (x - mu), axis=-1, keepdims=True)
    return jnp.maximum((x - mu) / jnp.sqrt(var + 1e-5) * g + b, 0.0)


def _enc_node_body(x_ref, W_ref, b_ref, g_ref, bb_ref, hv_ref, h1_ref):
    hv = jnp.dot(x_ref[...], W_ref[...], preferred_element_type=f32, precision=lax.Precision.HIGHEST) + b_ref[...]
    hv_ref[...] = hv
    h1_ref[...] = _ln_relu(hv, g_ref[...], bb_ref[...])


def _enc_edge_body(x_ref, W_ref, b_ref, he_ref):
    he_ref[...] = jnp.dot(x_ref[...], W_ref[...], preferred_element_type=f32, precision=lax.Precision.HIGHEST) + b_ref[...]


def _layer_body(accA_ref, accB_ref, h1_ref, hv_ref, W_ref, b_ref, g_ref, bb_ref,
                hvn_ref, h1n_ref):
    a0 = accA_ref[0]
    a1 = accB_ref[0]
    num = jnp.concatenate([a0[:, :64], a1[:, :64]], axis=1)
    den = jnp.concatenate([a0[:, 64:], a1[:, 64:]], axis=1)
    agg = jnp.where(den > 0.0, num / den, 0.0)
    hvn = (jnp.dot(h1_ref[...] + agg, W_ref[...], preferred_element_type=f32, precision=lax.Precision.HIGHEST)
           + b_ref[...] + hv_ref[...])
    hvn_ref[...] = hvn
    h1n_ref[...] = _ln_relu(hvn, g_ref[...], bb_ref[...])


def _readout_body(hv_ref,
                  Wa0, Wb0, lb0, Wp0, pb0, WihT0, WhhT0, bih0, bhh0,
                  Wa1, Wb1, lb1, Wp1, pb1, WihT1, WhhT1, bih1, bhh1,
                  W1_ref, b1_ref, W2_ref, b2_ref, out_ref):
    hv = hv_ref[...]
    g = jnp.sum(hv, axis=0, keepdims=True)
    steps = [
        (Wa0, Wb0, lb0, Wp0, pb0, WihT0, WhhT0, bih0, bhh0),
        (Wa1, Wb1, lb1, Wp1, pb1, WihT1, WhhT1, bih1, bhh1),
    ]
    for (Wa, Wb, lb, Wp, pb, WihT, WhhT, bih, bhh) in steps:
        rg = jnp.maximum(g, 0.0)
        s0 = jnp.sum(rg * Wa[...]) + lb[0, 0]
        z = jnp.dot(hv, Wb[...], preferred_element_type=f32, precision=lax.Precision.HIGHEST) + s0
        z = jnp.where(z > 0.0, z, 0.01 * z)
        mx = jnp.max(z)
        ex = jnp.exp(z - mx)
        aw = ex / jnp.sum(ex)
        wsum = jnp.sum(aw * hv, axis=0, keepdims=True)
        pr = jnp.dot(wsum, Wp[...], preferred_element_type=f32, precision=lax.Precision.HIGHEST) + pb[...]
        gr = jnp.where(pr > 0.0, pr, jnp.exp(jnp.minimum(pr, 0.0)) - 1.0)
        gi = jnp.dot(gr, WihT[...], preferred_element_type=f32, precision=lax.Precision.HIGHEST) + bih[...]
        gh = jnp.dot(g, WhhT[...], preferred_element_type=f32, precision=lax.Precision.HIGHEST) + bhh[...]
        r = jax.nn.sigmoid(gi[:, :H] + gh[:, :H])
        u = jax.nn.sigmoid(gi[:, H:2 * H] + gh[:, H:2 * H])
        c = jnp.tanh(gi[:, 2 * H:] + r * gh[:, 2 * H:])
        g = jnp.maximum((1.0 - u) * c + u * g, 0.0)
    h = jnp.maximum(jnp.dot(g, W1_ref[...], preferred_element_type=f32, precision=lax.Precision.HIGHEST)
                    + b1_ref[...], 0.0)
    out_ref[...] = jnp.dot(h, W2_ref[...], preferred_element_type=f32, precision=lax.Precision.HIGHEST) + b2_ref[...]


def _full(shape):
    nd = len(shape)
    return pl.BlockSpec(shape, lambda i: (0,) * nd)


def _rows(shape):
    nd = len(shape)
    return pl.BlockSpec(shape, lambda i: (i,) + (0,) * (nd - 1))


def _enc_node(node_feat, Wn, bn, g0, b0):
    return pl.pallas_call(
        _enc_node_body,
        grid=(N // RB,),
        in_specs=[_rows((RB, H)), _full((H, H)), _full((1, H)),
                  _full((1, H)), _full((1, H))],
        out_specs=[_rows((RB, H)), _rows((RB, H))],
        out_shape=[jax.ShapeDtypeStruct((N, H), f32),
                   jax.ShapeDtypeStruct((N, H), f32)],
    )(node_feat, Wn, bn, g0, b0)


def _enc_edge(efp, We, be):
    return pl.pallas_call(
        _enc_edge_body,
        grid=(EP // EB,),
        in_specs=[_rows((EB, DE)), _full((DE, H)), _full((1, H))],
        out_specs=[_rows((EB, H))],
        out_shape=[jax.ShapeDtypeStruct((EP, H), f32)],
    )(efp, We, be)


def _layer_dense(acc_out, h1, hv, Wl, bl, gN, bN):
    return pl.pallas_call(
        _layer_body,
        grid=(N // RB,),
        in_specs=[pl.BlockSpec((1, RB, H), lambda i: (0, i, 0)),
                  pl.BlockSpec((1, RB, H), lambda i: (1, i, 0)),
                  _rows((RB, H)), _rows((RB, H)), _full((H, H)),
                  _full((1, H)), _full((1, H)), _full((1, H))],
        out_specs=[_rows((RB, H)), _rows((RB, H))],
        out_shape=[jax.ShapeDtypeStruct((N, H), f32),
                   jax.ShapeDtypeStruct((N, H), f32)],
    )(acc_out, acc_out, h1, hv, Wl, bl, gN, bN)


def kernel(node_feat, edge_feat, edge_index, enc_node_W, enc_node_b, enc_edge_W, enc_edge_b, ln_g, ln_b, conv_W, conv_b, conv_beta, ro_logit_W, ro_logit_b, ro_proj_W, ro_proj_b, gru_Wih, gru_Whh, gru_bih, gru_bhh, out_W1, out_b1, out_W2, out_b2):
    src = edge_index[0]
    dst = edge_index[1]
    srcp = jnp.concatenate([src, jnp.zeros((EP - E,), jnp.int32)])
    dstp = jnp.concatenate([dst, jnp.full((EP - E,), N, jnp.int32)])
    efp = jnp.concatenate([edge_feat, jnp.zeros((EP - E, DE), f32)], axis=0)

    hv, h1 = _enc_node(node_feat, enc_node_W, enc_node_b.reshape(1, H),
                       ln_g[0].reshape(1, H), ln_b[0].reshape(1, H))
    (hef,) = _enc_edge(efp, enc_edge_W, enc_edge_b.reshape(1, H))

    for l in range(L):
        betav = jnp.broadcast_to(conv_beta[l].astype(f32), (16,))
        acc_out = _get_sc_kernel()(h1, hef, srcp, dstp, betav)
        ln_next = (l + 1) % L
        hv, h1 = _layer_dense(
            acc_out, h1, hv, conv_W[l], conv_b[l].reshape(1, H),
            ln_g[ln_next].reshape(1, H), ln_b[ln_next].reshape(1, H))

    WihT = jnp.transpose(gru_Wih, (0, 2, 1))
    WhhT = jnp.transpose(gru_Whh, (0, 2, 1))
    args = [hv]
    for t in range(T):
        args += [ro_logit_W[t, :H].reshape(1, H),
                 ro_logit_W[t, H:].reshape(H, 1),
                 ro_logit_b[t].reshape(1, 1),
                 ro_proj_W[t], ro_proj_b[t].reshape(1, H),
                 WihT[t], WhhT[t],
                 gru_bih[t].reshape(1, 3 * H), gru_bhh[t].reshape(1, 3 * H)]
    args += [out_W1, out_b1.reshape(1, 1024), out_W2, out_b2.reshape(1, 1)]
    o = pl.pallas_call(
        _readout_body,
        out_shape=jax.ShapeDtypeStruct((1, 1), f32),
    )(*args)
    return o.reshape(1)


# speedup vs baseline: 2.6524x; 1.3442x over previous
import functools

import jax
import jax.numpy as jnp
from jax import lax
from jax.experimental import pallas as pl
from jax.experimental.pallas import tpu as pltpu
from jax.experimental.pallas import tpu_sc as plsc

N = 10000
E = 160000
DE = 16
H = 128
L = 7
T = 2
EPS = 1e-7

NSUB = 16
CH = 64
EP = 163840
NCHUNK = EP // CH
CPT = NCHUNK // NSUB
NP = 10016
RB = 400
EB = 2048

f32 = jnp.float32


@functools.cache
def _get_sc_kernel():
    mesh = plsc.VectorSubcoreMesh(core_axis_name="c", subcore_axis_name="s")

    @functools.partial(
        pl.kernel,
        out_type=jax.ShapeDtypeStruct((2, NP, 128), f32),
        mesh=mesh,
        scratch_types=[
            pltpu.VMEM((CH,), jnp.int32),
            pltpu.VMEM((CH,), jnp.int32),
            pltpu.VMEM((CH,), jnp.int32),
            pltpu.VMEM((CH,), jnp.int32),
            pltpu.VMEM((CH,), jnp.int32),
            pltpu.VMEM((CH,), jnp.int32),
            pltpu.VMEM((CH, 128), f32),
            pltpu.VMEM((CH, 128), f32),
            pltpu.VMEM((CH, 128), f32),
            pltpu.VMEM((CH, 128), f32),
            pltpu.VMEM((CH, 128), f32),
            pltpu.VMEM((CH, 128), f32),
            pltpu.VMEM((16,), f32),
            pltpu.VMEM_SHARED((NP, 128), f32),
            pltpu.SemaphoreType.DMA,
            pltpu.SemaphoreType.DMA,
            pltpu.SemaphoreType.DMA,
            pltpu.SemaphoreType.DMA,
            pltpu.SemaphoreType.DMA,
            pltpu.SemaphoreType.DMA,
        ],
    )
    def _sc_layer_agg(h1, hef, srci, dsti, betav, acc_out,
                      idx_s0, idx_s1, idx_d0s, idx_d1s, idx_d0, idx_d1,
                      gbuf0, gbuf1, hebuf0, hebuf1, vals0, vals1, bbuf, acc,
                      gsem0, gsem1, hsem0, hsem1, ssem0, ssem1):
        cid = lax.axis_index("c")
        sid = lax.axis_index("s")

        zv = jnp.zeros((16,), f32)

        @pl.loop(0, 32)
        def _(i):
            for k in range(8):
                vals0[i, pl.ds(k * 16, 16)] = zv

        @pl.loop(sid, NP // 32, step=NSUB)
        def _(c):
            pltpu.sync_copy(vals0.at[pl.ds(0, 32)], acc.at[pl.ds(c * 32, 32)])

        pltpu.sync_copy(betav, bbuf)
        plsc.subcore_barrier()

        start = sid * CPT

        def issue(c, idx_s, idx_ds, gbuf, hebuf, gsem, hsem):
            base = c * CH
            pltpu.sync_copy(srci.at[pl.ds(base, CH)], idx_s)
            pltpu.sync_copy(dsti.at[pl.ds(base, CH)], idx_ds)
            pltpu.async_copy(h1.at[idx_s], gbuf, gsem)
            pltpu.async_copy(hef.at[pl.ds(base, CH)], hebuf, hsem)

        def drain(sem, dstbuf):
            pltpu.make_async_copy(hef.at[pl.ds(0, CH)], dstbuf, sem).wait()

        def compute(gbuf, hebuf, vals):
            bv = bbuf[pl.ds(0, 16)]
            off = cid * 64

            @pl.loop(0, CH)
            def _(i):
                for k in range(4):
                    g = gbuf[i, pl.ds(off + k * 16, 16)]
                    hh = hebuf[i, pl.ds(off + k * 16, 16)]
                    m = jnp.maximum(g + hh, 0.0) + EPS
                    w = jnp.exp(m * bv)
                    vals[i, pl.ds(k * 16, 16)] = m * w
                    vals[i, pl.ds(64 + k * 16, 16)] = w

        def step(j, c, idx_s, idx_ds, idx_d, gbuf, hebuf, vals, gsem, hsem,
                 ssem):
            drain(gsem, gbuf)
            drain(hsem, hebuf)

            @pl.when(j > 0)
            def _():
                drain(ssem, vals)

            for k in range(CH // 16):
                idx_d[pl.ds(k * 16, 16)] = idx_ds[pl.ds(k * 16, 16)]
            compute(gbuf, hebuf, vals)
            pltpu.async_copy(vals, acc.at[idx_d], ssem, add=True)

            @pl.when(j < CPT // 2 - 1)
            def _():
                issue(c + 2, idx_s, idx_ds, gbuf, hebuf, gsem, hsem)

        issue(start, idx_s0, idx_d0s, gbuf0, hebuf0, gsem0, hsem0)
        issue(start + 1, idx_s1, idx_d1s, gbuf1, hebuf1, gsem1, hsem1)

        @pl.loop(0, CPT // 2)
        def _(j):
            a = start + 2 * j
            step(j, a, idx_s0, idx_d0s, idx_d0, gbuf0, hebuf0, vals0,
                 gsem0, hsem0, ssem0)
            step(j, a + 1, idx_s1, idx_d1s, idx_d1, gbuf1, hebuf1, vals1,
                 gsem1, hsem1, ssem1)

        drain(ssem0, vals0)
        drain(ssem1, vals1)
        plsc.subcore_barrier()

        @pl.loop(sid, NP // 32, step=NSUB)
        def _(c):
            pltpu.sync_copy(acc.at[pl.ds(c * 32, 32)], vals0.at[pl.ds(0, 32)])
            pltpu.sync_copy(vals0.at[pl.ds(0, 32)],
                            acc_out.at[cid, pl.ds(c * 32, 32), :])

    return _sc_layer_agg


def _ln_relu(x, g, b):
    mu = jnp.mean(x, axis=-1, keepdims=True)
    var = jnp.mean((x - mu) * (x - mu), axis=-1, keepdims=True)
    return jnp.maximum((x - mu) / jnp.sqrt(var + 1e-5) * g + b, 0.0)


def _enc_node_body(x_ref, W_ref, b_ref, g_ref, bb_ref, hv_ref, h1_ref):
    hv = jnp.dot(x_ref[...], W_ref[...], preferred_element_type=f32, precision=lax.Precision.HIGHEST) + b_ref[...]
    hv_ref[...] = hv
    h1_ref[...] = _ln_relu(hv, g_ref[...], bb_ref[...])


def _enc_edge_body(x_ref, W_ref, b_ref, he_ref):
    he_ref[...] = jnp.dot(x_ref[...], W_ref[...], preferred_element_type=f32, precision=lax.Precision.HIGHEST) + b_ref[...]


def _layer_body(accA_ref, accB_ref, h1_ref, hv_ref, W_ref, b_ref, g_ref, bb_ref,
                hvn_ref, h1n_ref):
    a0 = accA_ref[0]
    a1 = accB_ref[0]
    num = jnp.concatenate([a0[:, :64], a1[:, :64]], axis=1)
    den = jnp.concatenate([a0[:, 64:], a1[:, 64:]], axis=1)
    agg = jnp.where(den > 0.0, num / den, 0.0)
    hvn = (jnp.dot(h1_ref[...] + agg, W_ref[...], preferred_element_type=f32, precision=lax.Precision.HIGHEST)
           + b_ref[...] + hv_ref[...])
    hvn_ref[...] = hvn
    h1n_ref[...] = _ln_relu(hvn, g_ref[...], bb_ref[...])


def _readout_body(hv_ref,
                  Wa0, Wb0, lb0, Wp0, pb0, WihT0, WhhT0, bih0, bhh0,
                  Wa1, Wb1, lb1, Wp1, pb1, WihT1, WhhT1, bih1, bhh1,
                  W1_ref, b1_ref, W2_ref, b2_ref, out_ref):
    hv = hv_ref[...]
    g = jnp.sum(hv, axis=0, keepdims=True)
    steps = [
        (Wa0, Wb0, lb0, Wp0, pb0, WihT0, WhhT0, bih0, bhh0),
        (Wa1, Wb1, lb1, Wp1, pb1, WihT1, WhhT1, bih1, bhh1),
    ]
    for (Wa, Wb, lb, Wp, pb, WihT, WhhT, bih, bhh) in steps:
        rg = jnp.maximum(g, 0.0)
        s0 = jnp.sum(rg * Wa[...]) + lb[0, 0]
        z = jnp.dot(hv, Wb[...], preferred_element_type=f32, precision=lax.Precision.HIGHEST) + s0
        z = jnp.where(z > 0.0, z, 0.01 * z)
        mx = jnp.max(z)
        ex = jnp.exp(z - mx)
        aw = ex / jnp.sum(ex)
        wsum = jnp.sum(aw * hv, axis=0, keepdims=True)
        pr = jnp.dot(wsum, Wp[...], preferred_element_type=f32, precision=lax.Precision.HIGHEST) + pb[...]
        gr = jnp.where(pr > 0.0, pr, jnp.exp(jnp.minimum(pr, 0.0)) - 1.0)
        gi = jnp.dot(gr, WihT[...], preferred_element_type=f32, precision=lax.Precision.HIGHEST) + bih[...]
        gh = jnp.dot(g, WhhT[...], preferred_element_type=f32, precision=lax.Precision.HIGHEST) + bhh[...]
        r = jax.nn.sigmoid(gi[:, :H] + gh[:, :H])
        u = jax.nn.sigmoid(gi[:, H:2 * H] + gh[:, H:2 * H])
        c = jnp.tanh(gi[:, 2 * H:] + r * gh[:, 2 * H:])
        g = jnp.maximum((1.0 - u) * c + u * g, 0.0)
    h = jnp.maximum(jnp.dot(g, W1_ref[...], preferred_element_type=f32, precision=lax.Precision.HIGHEST)
                    + b1_ref[...], 0.0)
    out_ref[...] = jnp.dot(h, W2_ref[...], preferred_element_type=f32, precision=lax.Precision.HIGHEST) + b2_ref[...]


def _full(shape):
    nd = len(shape)
    return pl.BlockSpec(shape, lambda i: (0,) * nd)


def _rows(shape):
    nd = len(shape)
    return pl.BlockSpec(shape, lambda i: (i,) + (0,) * (nd - 1))


def _enc_node(node_feat, Wn, bn, g0, b0):
    return pl.pallas_call(
        _enc_node_body,
        grid=(N // RB,),
        in_specs=[_rows((RB, H)), _full((H, H)), _full((1, H)),
                  _full((1, H)), _full((1, H))],
        out_specs=[_rows((RB, H)), _rows((RB, H))],
        out_shape=[jax.ShapeDtypeStruct((N, H), f32),
                   jax.ShapeDtypeStruct((N, H), f32)],
    )(node_feat, Wn, bn, g0, b0)


def _enc_edge(efp, We, be):
    return pl.pallas_call(
        _enc_edge_body,
        grid=(EP // EB,),
        in_specs=[_rows((EB, DE)), _full((DE, H)), _full((1, H))],
        out_specs=[_rows((EB, H))],
        out_shape=[jax.ShapeDtypeStruct((EP, H), f32)],
    )(efp, We, be)


def _layer_dense(acc_out, h1, hv, Wl, bl, gN, bN):
    return pl.pallas_call(
        _layer_body,
        grid=(N // RB,),
        in_specs=[pl.BlockSpec((1, RB, H), lambda i: (0, i, 0)),
                  pl.BlockSpec((1, RB, H), lambda i: (1, i, 0)),
                  _rows((RB, H)), _rows((RB, H)), _full((H, H)),
                  _full((1, H)), _full((1, H)), _full((1, H))],
        out_specs=[_rows((RB, H)), _rows((RB, H))],
        out_shape=[jax.ShapeDtypeStruct((N, H), f32),
                   jax.ShapeDtypeStruct((N, H), f32)],
    )(acc_out, acc_out, h1, hv, Wl, bl, gN, bN)


def kernel(node_feat, edge_feat, edge_index, enc_node_W, enc_node_b, enc_edge_W, enc_edge_b, ln_g, ln_b, conv_W, conv_b, conv_beta, ro_logit_W, ro_logit_b, ro_proj_W, ro_proj_b, gru_Wih, gru_Whh, gru_bih, gru_bhh, out_W1, out_b1, out_W2, out_b2):
    src = edge_index[0]
    dst = edge_index[1]
    srcp = jnp.concatenate([src, jnp.zeros((EP - E,), jnp.int32)])
    dstp = jnp.concatenate([dst, jnp.full((EP - E,), N, jnp.int32)])
    efp = jnp.concatenate([edge_feat, jnp.zeros((EP - E, DE), f32)], axis=0)

    hv, h1 = _enc_node(node_feat, enc_node_W, enc_node_b.reshape(1, H),
                       ln_g[0].reshape(1, H), ln_b[0].reshape(1, H))
    (hef,) = _enc_edge(efp, enc_edge_W, enc_edge_b.reshape(1, H))

    for l in range(L):
        betav = jnp.broadcast_to(conv_beta[l].astype(f32), (16,))
        acc_out = _get_sc_kernel()(h1, hef, srcp, dstp, betav)
        ln_next = (l + 1) % L
        hv, h1 = _layer_dense(
            acc_out, h1, hv, conv_W[l], conv_b[l].reshape(1, H),
            ln_g[ln_next].reshape(1, H), ln_b[ln_next].reshape(1, H))

    WihT = jnp.transpose(gru_Wih, (0, 2, 1))
    WhhT = jnp.transpose(gru_Whh, (0, 2, 1))
    args = [hv]
    for t in range(T):
        args += [ro_logit_W[t, :H].reshape(1, H),
                 ro_logit_W[t, H:].reshape(H, 1),
                 ro_logit_b[t].reshape(1, 1),
                 ro_proj_W[t], ro_proj_b[t].reshape(1, H),
                 WihT[t], WhhT[t],
                 gru_bih[t].reshape(1, 3 * H), gru_bhh[t].reshape(1, 3 * H)]
    args += [out_W1, out_b1.reshape(1, 1024), out_W2, out_b2.reshape(1, 1)]
    o = pl.pallas_call(
        _readout_body,
        out_shape=jax.ShapeDtypeStruct((1, 1), f32),
    )(*args)
    return o.reshape(1)


# speedup vs baseline: 2.9221x; 1.1017x over previous
import functools

import jax
import jax.numpy as jnp
from jax import lax
from jax.experimental import pallas as pl
from jax.experimental.pallas import tpu as pltpu
from jax.experimental.pallas import tpu_sc as plsc

N = 10000
E = 160000
DE = 16
H = 128
L = 7
T = 2
EPS = 1e-7

NSUB = 16
CH = 64
EP = 163840
NCHUNK = EP // CH
CPT = NCHUNK // NSUB
NP = 10016
RB = 400
EB = 2048

f32 = jnp.float32


@functools.cache
def _get_sc_kernel():
    mesh = plsc.VectorSubcoreMesh(core_axis_name="c", subcore_axis_name="s")

    @functools.partial(
        pl.kernel,
        out_type=jax.ShapeDtypeStruct((2, NP, 128), f32),
        mesh=mesh,
        scratch_types=[
            [pltpu.VMEM((CH,), jnp.int32) for _ in range(5)],
            [pltpu.VMEM((CH,), jnp.int32) for _ in range(5)],
            pltpu.VMEM((CH, 128), f32),
            pltpu.VMEM((CH, 128), f32),
            pltpu.VMEM((CH, 128), f32),
            pltpu.VMEM((CH, 128), f32),
            pltpu.VMEM((CH, 128), f32),
            pltpu.VMEM((CH, 128), f32),
            pltpu.VMEM((16,), f32),
            pltpu.VMEM_SHARED((NP, 128), f32),
            [pltpu.SemaphoreType.DMA for _ in range(4)],
            [pltpu.SemaphoreType.DMA for _ in range(4)],
        ],
    )
    def _sc_layer_agg(h1, hef, srci, dsti, betav, acc_out,
                      idx0, idx1, gbuf0, gbuf1, hebuf0, hebuf1,
                      vals0, vals1, bbuf, acc, sems0, sems1):
        cid = lax.axis_index("c")
        sid = lax.axis_index("s")

        zv = jnp.zeros((16,), f32)

        @pl.loop(0, 32)
        def _(i):
            for k in range(8):
                vals0[i, pl.ds(k * 16, 16)] = zv

        @pl.loop(sid, NP // 32, step=NSUB)
        def _(c):
            pltpu.sync_copy(vals0.at[pl.ds(0, 32)], acc.at[pl.ds(c * 32, 32)])

        pltpu.sync_copy(betav, bbuf)
        plsc.subcore_barrier()

        start = sid * CPT
        half = CPT // 2

        def veccopy(dst, srcr):
            for k in range(CH // 16):
                dst[pl.ds(k * 16, 16)] = srcr[pl.ds(k * 16, 16)]

        def drain(sem, dstbuf):
            pltpu.make_async_copy(hef.at[pl.ds(0, CH)], dstbuf, sem).wait()

        def stage_load(c, idx, isem):
            stage_s, _, stage_d, _, _ = idx
            base = c * CH
            pltpu.async_copy(srci.at[pl.ds(base, CH)], stage_s, isem)
            pltpu.async_copy(dsti.at[pl.ds(base, CH)], stage_d, isem)

        def gather_issue(c, idx, gbuf, hebuf, gsem, hsem):
            _, act_s, _, _, _ = idx
            base = c * CH
            pltpu.async_copy(h1.at[act_s], gbuf, gsem)
            pltpu.async_copy(hef.at[pl.ds(base, CH)], hebuf, hsem)

        def compute(gbuf, hebuf, vals):
            bv = bbuf[pl.ds(0, 16)]
            off = cid * 64

            @pl.loop(0, CH)
            def _(i):
                for k in range(4):
                    g = gbuf[i, pl.ds(off + k * 16, 16)]
                    hh = hebuf[i, pl.ds(off + k * 16, 16)]
                    m = jnp.maximum(g + hh, 0.0) + EPS
                    w = jnp.exp(m * bv)
                    vals[i, pl.ds(k * 16, 16)] = m * w
                    vals[i, pl.ds(64 + k * 16, 16)] = w

        def prologue(s, idx, gbuf, hebuf, sems):
            stage_s, act_s, stage_d, mid_d, act_d = idx
            gsem, hsem, ssem, isem = sems
            c0 = start + s
            base = c0 * CH
            pltpu.sync_copy(srci.at[pl.ds(base, CH)], stage_s)
            pltpu.sync_copy(dsti.at[pl.ds(base, CH)], stage_d)
            veccopy(act_s, stage_s)
            veccopy(mid_d, stage_d)
            gather_issue(c0, idx, gbuf, hebuf, gsem, hsem)
            stage_load(c0 + 2, idx, isem)

        def step(j, s, idx, gbuf, hebuf, vals, sems):
            stage_s, act_s, stage_d, mid_d, act_d = idx
            gsem, hsem, ssem, isem = sems
            c = start + 2 * j + s
            drain(gsem, gbuf)
            drain(hsem, hebuf)

            @pl.when(j > 0)
            def _():
                drain(ssem, vals)

            veccopy(act_d, mid_d)

            @pl.when(j < half - 1)
            def _():
                pltpu.make_async_copy(srci.at[pl.ds(0, CH)], stage_s, isem).wait()
                pltpu.make_async_copy(srci.at[pl.ds(0, CH)], stage_d, isem).wait()
                veccopy(act_s, stage_s)
                veccopy(mid_d, stage_d)
                gather_issue(c + 2, idx, gbuf, hebuf, gsem, hsem)

            @pl.when(j < half - 2)
            def _():
                stage_load(c + 4, idx, isem)

            compute(gbuf, hebuf, vals)
            pltpu.async_copy(vals, acc.at[act_d], ssem, add=True)

        prologue(0, idx0, gbuf0, hebuf0, sems0)
        prologue(1, idx1, gbuf1, hebuf1, sems1)

        @pl.loop(0, half)
        def _(j):
            step(j, 0, idx0, gbuf0, hebuf0, vals0, sems0)
            step(j, 1, idx1, gbuf1, hebuf1, vals1, sems1)

        drain(sems0[2], vals0)
        drain(sems1[2], vals1)
        plsc.subcore_barrier()

        @pl.loop(sid, NP // 32, step=NSUB)
        def _(c):
            pltpu.sync_copy(acc.at[pl.ds(c * 32, 32)], vals0.at[pl.ds(0, 32)])
            pltpu.sync_copy(vals0.at[pl.ds(0, 32)],
                            acc_out.at[cid, pl.ds(c * 32, 32), :])

    return _sc_layer_agg


def _ln_relu(x, g, b):
    mu = jnp.mean(x, axis=-1, keepdims=True)
    var = jnp.mean((x - mu) * (x - mu), axis=-1, keepdims=True)
    return jnp.maximum((x - mu) / jnp.sqrt(var + 1e-5) * g + b, 0.0)


def _enc_node_body(x_ref, W_ref, b_ref, g_ref, bb_ref, hv_ref, h1_ref):
    hv = jnp.dot(x_ref[...], W_ref[...], preferred_element_type=f32, precision=lax.Precision.HIGHEST) + b_ref[...]
    hv_ref[...] = hv
    h1_ref[...] = _ln_relu(hv, g_ref[...], bb_ref[...])


def _enc_edge_body(x_ref, W_ref, b_ref, he_ref):
    he_ref[...] = jnp.dot(x_ref[...], W_ref[...], preferred_element_type=f32, precision=lax.Precision.HIGHEST) + b_ref[...]


def _layer_body(accA_ref, accB_ref, h1_ref, hv_ref, W_ref, b_ref, g_ref, bb_ref,
                hvn_ref, h1n_ref):
    a0 = accA_ref[0]
    a1 = accB_ref[0]
    num = jnp.concatenate([a0[:, :64], a1[:, :64]], axis=1)
    den = jnp.concatenate([a0[:, 64:], a1[:, 64:]], axis=1)
    agg = jnp.where(den > 0.0, num / den, 0.0)
    hvn = (jnp.dot(h1_ref[...] + agg, W_ref[...], preferred_element_type=f32, precision=lax.Precision.HIGHEST)
           + b_ref[...] + hv_ref[...])
    hvn_ref[...] = hvn
    h1n_ref[...] = _ln_relu(hvn, g_ref[...], bb_ref[...])


def _readout_body(hv_ref,
                  Wa0, Wb0, lb0, Wp0, pb0, WihT0, WhhT0, bih0, bhh0,
                  Wa1, Wb1, lb1, Wp1, pb1, WihT1, WhhT1, bih1, bhh1,
                  W1_ref, b1_ref, W2_ref, b2_ref, out_ref):
    hv = hv_ref[...]
    g = jnp.sum(hv, axis=0, keepdims=True)
    steps = [
        (Wa0, Wb0, lb0, Wp0, pb0, WihT0, WhhT0, bih0, bhh0),
        (Wa1, Wb1, lb1, Wp1, pb1, WihT1, WhhT1, bih1, bhh1),
    ]
    for (Wa, Wb, lb, Wp, pb, WihT, WhhT, bih, bhh) in steps:
        rg = jnp.maximum(g, 0.0)
        s0 = jnp.sum(rg * Wa[...]) + lb[0, 0]
        z = jnp.dot(hv, Wb[...], preferred_element_type=f32, precision=lax.Precision.HIGHEST) + s0
        z = jnp.where(z > 0.0, z, 0.01 * z)
        mx = jnp.max(z)
        ex = jnp.exp(z - mx)
        aw = ex / jnp.sum(ex)
        wsum = jnp.sum(aw * hv, axis=0, keepdims=True)
        pr = jnp.dot(wsum, Wp[...], preferred_element_type=f32, precision=lax.Precision.HIGHEST) + pb[...]
        gr = jnp.where(pr > 0.0, pr, jnp.exp(jnp.minimum(pr, 0.0)) - 1.0)
        gi = jnp.dot(gr, WihT[...], preferred_element_type=f32, precision=lax.Precision.HIGHEST) + bih[...]
        gh = jnp.dot(g, WhhT[...], preferred_element_type=f32, precision=lax.Precision.HIGHEST) + bhh[...]
        r = jax.nn.sigmoid(gi[:, :H] + gh[:, :H])
        u = jax.nn.sigmoid(gi[:, H:2 * H] + gh[:, H:2 * H])
        c = jnp.tanh(gi[:, 2 * H:] + r * gh[:, 2 * H:])
        g = jnp.maximum((1.0 - u) * c + u * g, 0.0)
    h = jnp.maximum(jnp.dot(g, W1_ref[...], preferred_element_type=f32, precision=lax.Precision.HIGHEST)
                    + b1_ref[...], 0.0)
    out_ref[...] = jnp.dot(h, W2_ref[...], preferred_element_type=f32, precision=lax.Precision.HIGHEST) + b2_ref[...]


def _full(shape):
    nd = len(shape)
    return pl.BlockSpec(shape, lambda i: (0,) * nd)


def _rows(shape):
    nd = len(shape)
    return pl.BlockSpec(shape, lambda i: (i,) + (0,) * (nd - 1))


def _enc_node(node_feat, Wn, bn, g0, b0):
    return pl.pallas_call(
        _enc_node_body,
        grid=(N // RB,),
        in_specs=[_rows((RB, H)), _full((H, H)), _full((1, H)),
                  _full((1, H)), _full((1, H))],
        out_specs=[_rows((RB, H)), _rows((RB, H))],
        out_shape=[jax.ShapeDtypeStruct((N, H), f32),
                   jax.ShapeDtypeStruct((N, H), f32)],
    )(node_feat, Wn, bn, g0, b0)


def _enc_edge(efp, We, be):
    return pl.pallas_call(
        _enc_edge_body,
        grid=(EP // EB,),
        in_specs=[_rows((EB, DE)), _full((DE, H)), _full((1, H))],
        out_specs=[_rows((EB, H))],
        out_shape=[jax.ShapeDtypeStruct((EP, H), f32)],
    )(efp, We, be)


def _layer_dense(acc_out, h1, hv, Wl, bl, gN, bN):
    return pl.pallas_call(
        _layer_body,
        grid=(N // RB,),
        in_specs=[pl.BlockSpec((1, RB, H), lambda i: (0, i, 0)),
                  pl.BlockSpec((1, RB, H), lambda i: (1, i, 0)),
                  _rows((RB, H)), _rows((RB, H)), _full((H, H)),
                  _full((1, H)), _full((1, H)), _full((1, H))],
        out_specs=[_rows((RB, H)), _rows((RB, H))],
        out_shape=[jax.ShapeDtypeStruct((N, H), f32),
                   jax.ShapeDtypeStruct((N, H), f32)],
    )(acc_out, acc_out, h1, hv, Wl, bl, gN, bN)


def kernel(node_feat, edge_feat, edge_index, enc_node_W, enc_node_b, enc_edge_W, enc_edge_b, ln_g, ln_b, conv_W, conv_b, conv_beta, ro_logit_W, ro_logit_b, ro_proj_W, ro_proj_b, gru_Wih, gru_Whh, gru_bih, gru_bhh, out_W1, out_b1, out_W2, out_b2):
    src = edge_index[0]
    dst = edge_index[1]
    srcp = jnp.concatenate([src, jnp.zeros((EP - E,), jnp.int32)])
    dstp = jnp.concatenate([dst, jnp.full((EP - E,), N, jnp.int32)])
    efp = jnp.concatenate([edge_feat, jnp.zeros((EP - E, DE), f32)], axis=0)

    hv, h1 = _enc_node(node_feat, enc_node_W, enc_node_b.reshape(1, H),
                       ln_g[0].reshape(1, H), ln_b[0].reshape(1, H))
    (hef,) = _enc_edge(efp, enc_edge_W, enc_edge_b.reshape(1, H))

    for l in range(L):
        betav = jnp.broadcast_to(conv_beta[l].astype(f32), (16,))
        acc_out = _get_sc_kernel()(h1, hef, srcp, dstp, betav)
        ln_next = (l + 1) % L
        hv, h1 = _layer_dense(
            acc_out, h1, hv, conv_W[l], conv_b[l].reshape(1, H),
            ln_g[ln_next].reshape(1, H), ln_b[ln_next].reshape(1, H))

    WihT = jnp.transpose(gru_Wih, (0, 2, 1))
    WhhT = jnp.transpose(gru_Whh, (0, 2, 1))
    args = [hv]
    for t in range(T):
        args += [ro_logit_W[t, :H].reshape(1, H),
                 ro_logit_W[t, H:].reshape(H, 1),
                 ro_logit_b[t].reshape(1, 1),
                 ro_proj_W[t], ro_proj_b[t].reshape(1, H),
                 WihT[t], WhhT[t],
                 gru_bih[t].reshape(1, 3 * H), gru_bhh[t].reshape(1, 3 * H)]
    args += [out_W1, out_b1.reshape(1, 1024), out_W2, out_b2.reshape(1, 1)]
    o = pl.pallas_call(
        _readout_body,
        out_shape=jax.ShapeDtypeStruct((1, 1), f32),
    )(*args)
    return o.reshape(1)


# speedup vs baseline: 4.7429x; 1.6231x over previous
import functools

import jax
import jax.numpy as jnp
from jax import lax
from jax.experimental import pallas as pl
from jax.experimental.pallas import tpu as pltpu
from jax.experimental.pallas import tpu_sc as plsc

N = 10000
E = 160000
DE = 16
H = 128
L = 7
T = 2
EPS = 1e-7

NSUB = 16
CH = 64
EP = 163840
NCHUNK = EP // CH
CPT = NCHUNK // NSUB
NP = 10016
RB = 400
EB = 2048

f32 = jnp.float32


@functools.cache
def _get_sc_kernel():
    mesh = plsc.VectorSubcoreMesh(core_axis_name="c", subcore_axis_name="s")

    @functools.partial(
        pl.kernel,
        out_type=jax.ShapeDtypeStruct((2, NP, 128), f32),
        mesh=mesh,
        scratch_types=[
            [pltpu.VMEM((CH,), jnp.int32) for _ in range(5)],
            [pltpu.VMEM((CH,), jnp.int32) for _ in range(5)],
            pltpu.VMEM((CH, 128), f32),
            pltpu.VMEM((CH, 128), f32),
            pltpu.VMEM((CH, 128), f32),
            pltpu.VMEM((CH, 128), f32),
            pltpu.VMEM((CH, 128), f32),
            pltpu.VMEM((CH, 128), f32),
            pltpu.VMEM((16,), f32),
            pltpu.VMEM_SHARED((NP, 128), f32),
            [pltpu.SemaphoreType.DMA for _ in range(4)],
            [pltpu.SemaphoreType.DMA for _ in range(4)],
        ],
    )
    def _sc_layer_agg(h1, hef, srci, dsti, betav, acc_out,
                      idx0, idx1, gbuf0, gbuf1, hebuf0, hebuf1,
                      vals0, vals1, bbuf, acc, sems0, sems1):
        cid = lax.axis_index("c")
        sid = lax.axis_index("s")

        zv = jnp.zeros((16,), f32)

        @pl.loop(0, 32)
        def _(i):
            for k in range(8):
                vals0[i, pl.ds(k * 16, 16)] = zv

        @pl.loop(sid, NP // 32, step=NSUB)
        def _(c):
            pltpu.sync_copy(vals0.at[pl.ds(0, 32)], acc.at[pl.ds(c * 32, 32)])

        pltpu.sync_copy(betav, bbuf)
        plsc.subcore_barrier()

        start = sid * CPT
        half = CPT // 2

        def veccopy(dst, srcr):
            for k in range(CH // 16):
                dst[pl.ds(k * 16, 16)] = srcr[pl.ds(k * 16, 16)]

        def drain(sem, dstbuf):
            pltpu.make_async_copy(hef.at[pl.ds(0, CH)], dstbuf, sem).wait()

        def stage_load(c, idx, isem):
            stage_s, _, stage_d, _, _ = idx
            base = c * CH
            pltpu.async_copy(srci.at[pl.ds(base, CH)], stage_s, isem)
            pltpu.async_copy(dsti.at[pl.ds(base, CH)], stage_d, isem)

        def gather_issue(c, idx, gbuf, hebuf, gsem, hsem):
            _, act_s, _, _, _ = idx
            base = c * CH
            pltpu.async_copy(h1.at[act_s], gbuf, gsem)
            pltpu.async_copy(hef.at[pl.ds(base, CH)], hebuf, hsem)

        def compute(gbuf, hebuf, vals):
            bv = bbuf[pl.ds(0, 16)]
            off = cid * 64

            @plsc.parallel_loop(0, CH, unroll=4)
            def _(i):
                for k in range(4):
                    g = gbuf[i, pl.ds(off + k * 16, 16)]
                    hh = hebuf[i, pl.ds(off + k * 16, 16)]
                    m = jnp.maximum(g + hh, 0.0) + EPS
                    w = jnp.exp(m * bv)
                    vals[i, pl.ds(k * 16, 16)] = m * w
                    vals[i, pl.ds(64 + k * 16, 16)] = w

        def prologue(s, idx, gbuf, hebuf, sems):
            stage_s, act_s, stage_d, mid_d, act_d = idx
            gsem, hsem, ssem, isem = sems
            c0 = start + s
            base = c0 * CH
            pltpu.sync_copy(srci.at[pl.ds(base, CH)], stage_s)
            pltpu.sync_copy(dsti.at[pl.ds(base, CH)], stage_d)
            veccopy(act_s, stage_s)
            veccopy(mid_d, stage_d)
            gather_issue(c0, idx, gbuf, hebuf, gsem, hsem)
            stage_load(c0 + 2, idx, isem)

        def step(j, s, idx, gbuf, hebuf, vals, sems):
            stage_s, act_s, stage_d, mid_d, act_d = idx
            gsem, hsem, ssem, isem = sems
            c = start + 2 * j + s
            drain(gsem, gbuf)
            drain(hsem, hebuf)

            @pl.when(j > 0)
            def _():
                drain(ssem, vals)

            veccopy(act_d, mid_d)

            @pl.when(j < half - 1)
            def _():
                pltpu.make_async_copy(srci.at[pl.ds(0, CH)], stage_s, isem).wait()
                pltpu.make_async_copy(srci.at[pl.ds(0, CH)], stage_d, isem).wait()
                veccopy(act_s, stage_s)
                veccopy(mid_d, stage_d)
                gather_issue(c + 2, idx, gbuf, hebuf, gsem, hsem)

            @pl.when(j < half - 2)
            def _():
                stage_load(c + 4, idx, isem)

            compute(gbuf, hebuf, vals)
            pltpu.async_copy(vals, acc.at[act_d], ssem, add=True)

        prologue(0, idx0, gbuf0, hebuf0, sems0)
        prologue(1, idx1, gbuf1, hebuf1, sems1)

        @pl.loop(0, half)
        def _(j):
            step(j, 0, idx0, gbuf0, hebuf0, vals0, sems0)
            step(j, 1, idx1, gbuf1, hebuf1, vals1, sems1)

        drain(sems0[2], vals0)
        drain(sems1[2], vals1)
        plsc.subcore_barrier()

        @pl.loop(sid, NP // 32, step=NSUB)
        def _(c):
            pltpu.sync_copy(acc.at[pl.ds(c * 32, 32)], vals0.at[pl.ds(0, 32)])
            pltpu.sync_copy(vals0.at[pl.ds(0, 32)],
                            acc_out.at[cid, pl.ds(c * 32, 32), :])

    return _sc_layer_agg


def _ln_relu(x, g, b):
    mu = jnp.mean(x, axis=-1, keepdims=True)
    var = jnp.mean((x - mu) * (x - mu), axis=-1, keepdims=True)
    return jnp.maximum((x - mu) / jnp.sqrt(var + 1e-5) * g + b, 0.0)


def _enc_node_body(x_ref, W_ref, b_ref, g_ref, bb_ref, hv_ref, h1_ref):
    hv = jnp.dot(x_ref[...], W_ref[...], preferred_element_type=f32, precision=lax.Precision.HIGHEST) + b_ref[...]
    hv_ref[...] = hv
    h1_ref[...] = _ln_relu(hv, g_ref[...], bb_ref[...])


def _enc_edge_body(x_ref, W_ref, b_ref, he_ref):
    he_ref[...] = jnp.dot(x_ref[...], W_ref[...], preferred_element_type=f32, precision=lax.Precision.HIGHEST) + b_ref[...]


def _layer_body(accA_ref, accB_ref, h1_ref, hv_ref, W_ref, b_ref, g_ref, bb_ref,
                hvn_ref, h1n_ref):
    a0 = accA_ref[0]
    a1 = accB_ref[0]
    num = jnp.concatenate([a0[:, :64], a1[:, :64]], axis=1)
    den = jnp.concatenate([a0[:, 64:], a1[:, 64:]], axis=1)
    agg = jnp.where(den > 0.0, num / den, 0.0)
    hvn = (jnp.dot(h1_ref[...] + agg, W_ref[...], preferred_element_type=f32, precision=lax.Precision.HIGHEST)
           + b_ref[...] + hv_ref[...])
    hvn_ref[...] = hvn
    h1n_ref[...] = _ln_relu(hvn, g_ref[...], bb_ref[...])


def _readout_body(hv_ref,
                  Wa0, Wb0, lb0, Wp0, pb0, WihT0, WhhT0, bih0, bhh0,
                  Wa1, Wb1, lb1, Wp1, pb1, WihT1, WhhT1, bih1, bhh1,
                  W1_ref, b1_ref, W2_ref, b2_ref, out_ref):
    hv = hv_ref[...]
    g = jnp.sum(hv, axis=0, keepdims=True)
    steps = [
        (Wa0, Wb0, lb0, Wp0, pb0, WihT0, WhhT0, bih0, bhh0),
        (Wa1, Wb1, lb1, Wp1, pb1, WihT1, WhhT1, bih1, bhh1),
    ]
    for (Wa, Wb, lb, Wp, pb, WihT, WhhT, bih, bhh) in steps:
        rg = jnp.maximum(g, 0.0)
        s0 = jnp.sum(rg * Wa[...]) + lb[0, 0]
        z = jnp.dot(hv, Wb[...], preferred_element_type=f32, precision=lax.Precision.HIGHEST) + s0
        z = jnp.where(z > 0.0, z, 0.01 * z)
        mx = jnp.max(z)
        ex = jnp.exp(z - mx)
        aw = ex / jnp.sum(ex)
        wsum = jnp.sum(aw * hv, axis=0, keepdims=True)
        pr = jnp.dot(wsum, Wp[...], preferred_element_type=f32, precision=lax.Precision.HIGHEST) + pb[...]
        gr = jnp.where(pr > 0.0, pr, jnp.exp(jnp.minimum(pr, 0.0)) - 1.0)
        gi = jnp.dot(gr, WihT[...], preferred_element_type=f32, precision=lax.Precision.HIGHEST) + bih[...]
        gh = jnp.dot(g, WhhT[...], preferred_element_type=f32, precision=lax.Precision.HIGHEST) + bhh[...]
        r = jax.nn.sigmoid(gi[:, :H] + gh[:, :H])
        u = jax.nn.sigmoid(gi[:, H:2 * H] + gh[:, H:2 * H])
        c = jnp.tanh(gi[:, 2 * H:] + r * gh[:, 2 * H:])
        g = jnp.maximum((1.0 - u) * c + u * g, 0.0)
    h = jnp.maximum(jnp.dot(g, W1_ref[...], preferred_element_type=f32, precision=lax.Precision.HIGHEST)
                    + b1_ref[...], 0.0)
    out_ref[...] = jnp.dot(h, W2_ref[...], preferred_element_type=f32, precision=lax.Precision.HIGHEST) + b2_ref[...]


def _full(shape):
    nd = len(shape)
    return pl.BlockSpec(shape, lambda i: (0,) * nd)


def _rows(shape):
    nd = len(shape)
    return pl.BlockSpec(shape, lambda i: (i,) + (0,) * (nd - 1))


def _enc_node(node_feat, Wn, bn, g0, b0):
    return pl.pallas_call(
        _enc_node_body,
        grid=(N // RB,),
        in_specs=[_rows((RB, H)), _full((H, H)), _full((1, H)),
                  _full((1, H)), _full((1, H))],
        out_specs=[_rows((RB, H)), _rows((RB, H))],
        out_shape=[jax.ShapeDtypeStruct((N, H), f32),
                   jax.ShapeDtypeStruct((N, H), f32)],
    )(node_feat, Wn, bn, g0, b0)


def _enc_edge(efp, We, be):
    return pl.pallas_call(
        _enc_edge_body,
        grid=(EP // EB,),
        in_specs=[_rows((EB, DE)), _full((DE, H)), _full((1, H))],
        out_specs=[_rows((EB, H))],
        out_shape=[jax.ShapeDtypeStruct((EP, H), f32)],
    )(efp, We, be)


def _layer_dense(acc_out, h1, hv, Wl, bl, gN, bN):
    return pl.pallas_call(
        _layer_body,
        grid=(N // RB,),
        in_specs=[pl.BlockSpec((1, RB, H), lambda i: (0, i, 0)),
                  pl.BlockSpec((1, RB, H), lambda i: (1, i, 0)),
                  _rows((RB, H)), _rows((RB, H)), _full((H, H)),
                  _full((1, H)), _full((1, H)), _full((1, H))],
        out_specs=[_rows((RB, H)), _rows((RB, H))],
        out_shape=[jax.ShapeDtypeStruct((N, H), f32),
                   jax.ShapeDtypeStruct((N, H), f32)],
    )(acc_out, acc_out, h1, hv, Wl, bl, gN, bN)


def kernel(node_feat, edge_feat, edge_index, enc_node_W, enc_node_b, enc_edge_W, enc_edge_b, ln_g, ln_b, conv_W, conv_b, conv_beta, ro_logit_W, ro_logit_b, ro_proj_W, ro_proj_b, gru_Wih, gru_Whh, gru_bih, gru_bhh, out_W1, out_b1, out_W2, out_b2):
    src = edge_index[0]
    dst = edge_index[1]
    srcp = jnp.concatenate([src, jnp.zeros((EP - E,), jnp.int32)])
    dstp = jnp.concatenate([dst, jnp.full((EP - E,), N, jnp.int32)])
    efp = jnp.concatenate([edge_feat, jnp.zeros((EP - E, DE), f32)], axis=0)

    hv, h1 = _enc_node(node_feat, enc_node_W, enc_node_b.reshape(1, H),
                       ln_g[0].reshape(1, H), ln_b[0].reshape(1, H))
    (hef,) = _enc_edge(efp, enc_edge_W, enc_edge_b.reshape(1, H))

    for l in range(L):
        betav = jnp.broadcast_to(conv_beta[l].astype(f32), (16,))
        acc_out = _get_sc_kernel()(h1, hef, srcp, dstp, betav)
        ln_next = (l + 1) % L
        hv, h1 = _layer_dense(
            acc_out, h1, hv, conv_W[l], conv_b[l].reshape(1, H),
            ln_g[ln_next].reshape(1, H), ln_b[ln_next].reshape(1, H))

    WihT = jnp.transpose(gru_Wih, (0, 2, 1))
    WhhT = jnp.transpose(gru_Whh, (0, 2, 1))
    args = [hv]
    for t in range(T):
        args += [ro_logit_W[t, :H].reshape(1, H),
                 ro_logit_W[t, H:].reshape(H, 1),
                 ro_logit_b[t].reshape(1, 1),
                 ro_proj_W[t], ro_proj_b[t].reshape(1, H),
                 WihT[t], WhhT[t],
                 gru_bih[t].reshape(1, 3 * H), gru_bhh[t].reshape(1, 3 * H)]
    args += [out_W1, out_b1.reshape(1, 1024), out_W2, out_b2.reshape(1, 1)]
    o = pl.pallas_call(
        _readout_body,
        out_shape=jax.ShapeDtypeStruct((1, 1), f32),
    )(*args)
    return o.reshape(1)
